# baseline (speedup 1.0000x reference)
import jax
import jax.numpy as jnp
from jax import lax
from jax.experimental import pallas as pl
from jax.experimental.pallas import tpu as pltpu
from jax.experimental.pallas import tpu_sc as plsc

_BS = 8
_D = 32
_K = 8
_S = 4
_OUT = (_D - _K) // _S + 1
_NWIN = _OUT * _OUT
_GROUPS = _BS * _OUT
_NC, _NS = 2, 16
_WORKERS = _NC * _NS
_ROUNDS = -(-_GROUPS // _WORKERS)


def _sc_body(x_hbm, out_hbm, buf_v, stage_v, sem0):
    wid = lax.axis_index("s") * _NC + lax.axis_index("c")
    iota = lax.iota(jnp.int32, 16)
    coffs = [32 * ((16 * gg + iota) >> 3) + ((16 * gg + iota) & 7)
             for gg in range(4)]

    def fetch_block(g):
        b = g // _OUT
        wi = g % _OUT
        pltpu.sync_copy(
            x_hbm.at[b, pl.ds(wi * 128, 256), pl.ds(wi * 128, 256)], buf_v)

    def extract_group(g, stage_v, sem):
        for wj in range(7):
            colvecs = [c + 4 * wj for c in coffs]

            @plsc.parallel_loop(0, 64, unroll=4)
            def body(i, wj=wj, colvecs=colvecs):
                row = 4 * wj + 32 * (i >> 3) + (i & 7)
                rowvec = jnp.full((16,), row, jnp.int32)
                for gg in range(4):
                    v = plsc.load_gather(buf_v, [rowvec, colvecs[gg]])
                    stage_v[wj, pl.ds(i * 64 + gg * 16, 16)] = v
        return pltpu.async_copy(stage_v, out_hbm.at[g], sem)

    g0 = wid
    g1 = wid + _WORKERS
    fetch_block(g0)
    cp0 = extract_group(g0, stage_v, sem0)

    @pl.when(g1 < _GROUPS)
    def _():
        fetch_block(g1)
        cp0.wait()
        cp1 = extract_group(g1, stage_v, sem0)
        cp1.wait()

    @pl.when(g1 >= _GROUPS)
    def _():
        cp0.wait()


def kernel(x):
    run = pl.kernel(
        _sc_body,
        out_type=jax.ShapeDtypeStruct((_GROUPS, _OUT, _K * _K * _K * _K),
                                      jnp.float32),
        mesh=plsc.VectorSubcoreMesh(core_axis_name="c", subcore_axis_name="s"),
        compiler_params=pltpu.CompilerParams(use_tc_tiling_on_sc=True, needs_layout_passes=False, skip_device_barrier=True, disable_bounds_checks=True),
        scratch_types=[
            pltpu.VMEM((256, 256), jnp.float32),
            pltpu.VMEM((_OUT, _K * _K * _K * _K), jnp.float32),
            pltpu.SemaphoreType.DMA,
        ],
    )
    out = run(x)
    return (out.reshape(_BS, _NWIN, _K * _K, _K * _K),)


# speedup vs baseline: 1.3028x; 1.3028x over previous
import jax
import jax.numpy as jnp
from jax import lax
from jax.experimental import pallas as pl
from jax.experimental.pallas import tpu as pltpu
from jax.experimental.pallas import tpu_sc as plsc

_BS = 8
_D = 32
_K = 8
_S = 4
_OUT = (_D - _K) // _S + 1
_NWIN = _OUT * _OUT
_GROUPS = _BS * _OUT
_NC, _NS = 2, 16
_WORKERS = _NC * _NS
_ROUNDS = -(-_GROUPS // _WORKERS)


def _sc_body(x_hbm, out_hbm, buf_v, stage_v, sem0):
    wid = lax.axis_index("s") * _NC + lax.axis_index("c")
    iota = lax.iota(jnp.int32, 16)
    coffs = [32 * ((16 * gg + iota) >> 3) + ((16 * gg + iota) & 7)
             for gg in range(4)]

    def fetch_block(g):
        b = g // _OUT
        wi = g % _OUT
        pltpu.sync_copy(
            x_hbm.at[b, pl.ds(wi * 128, 256), pl.ds(wi * 128, 256)], buf_v)

    def extract_group(g, stage_v, sem):
        for wj in range(7):
            colvecs = [c + 4 * wj for c in coffs]

            @plsc.parallel_loop(0, 64, unroll=4)
            def body(i, wj=wj, colvecs=colvecs):
                row = 4 * wj + 32 * (i >> 3) + (i & 7)
                rowvec = jnp.full((16,), row, jnp.int32)
                for gg in range(4):
                    v = plsc.load_gather(buf_v, [rowvec, colvecs[gg]])
                    stage_v[wj, i, pl.ds(gg * 16, 16)] = v
        return pltpu.async_copy(stage_v, out_hbm.at[pl.ds(g * _OUT, _OUT)], sem)

    g0 = wid
    g1 = wid + _WORKERS
    fetch_block(g0)
    cp0 = extract_group(g0, stage_v, sem0)

    @pl.when(g1 < _GROUPS)
    def _():
        fetch_block(g1)
        cp0.wait()
        cp1 = extract_group(g1, stage_v, sem0)
        cp1.wait()

    @pl.when(g1 >= _GROUPS)
    def _():
        cp0.wait()


def kernel(x):
    run = pl.kernel(
        _sc_body,
        out_type=jax.ShapeDtypeStruct((_BS * _NWIN, _K * _K, _K * _K),
                                      jnp.float32),
        mesh=plsc.VectorSubcoreMesh(core_axis_name="c", subcore_axis_name="s"),
        compiler_params=pltpu.CompilerParams(use_tc_tiling_on_sc=True, needs_layout_passes=False, skip_device_barrier=True, disable_bounds_checks=True),
        scratch_types=[
            pltpu.VMEM((256, 256), jnp.float32),
            pltpu.VMEM((_OUT, _K * _K, _K * _K), jnp.float32),
            pltpu.SemaphoreType.DMA,
        ],
    )
    out = run(x)
    return (out.reshape(_BS, _NWIN, _K * _K, _K * _K),)
